# baseline (speedup 1.0000x reference)
import functools

import jax
import jax.numpy as jnp
from jax import lax
from jax.experimental import pallas as pl
from jax.experimental.pallas import tpu as pltpu
from jax.experimental.pallas import tpu_sc as plsc

B_TOK = 4096
T_TOK = 200
EMB = 64
NW = 32
B = B_TOK * T_TOK
B_PER_W = B // NW
CH = 128
NCH = B_PER_W // CH

_NC = 2
_MESH = plsc.VectorSubcoreMesh(core_axis_name="c", subcore_axis_name="s")


@functools.partial(
    pl.kernel,
    mesh=_MESH,
    out_type=jax.ShapeDtypeStruct((B, EMB), jnp.float32),
    scratch_types=[
        pltpu.VMEM((NCH, CH), jnp.int32),
        pltpu.VMEM((CH, EMB), jnp.float32),
        pltpu.SemaphoreType.DMA,
    ],
    compiler_params=pltpu.CompilerParams(use_tc_tiling_on_sc=False),
)
def _gather_kernel(tok_hbm, table_hbm, out_hbm, idx_v, rows_v, gsem):
    wid = lax.axis_index("s") * _NC + lax.axis_index("c")
    base = wid * B_PER_W
    pltpu.sync_copy(tok_hbm.at[wid], idx_v)

    def chunk(j, carry):
        pltpu.async_copy(table_hbm.at[idx_v.at[j]], rows_v, gsem).wait()
        pltpu.sync_copy(rows_v, out_hbm.at[pl.ds(base + j * CH, CH)])
        return carry

    lax.fori_loop(0, NCH, chunk, 0)


def kernel(tokens, table):
    tok = tokens.reshape(NW, NCH, CH)
    out = _gather_kernel(tok, table)
    return out.reshape(B_TOK, T_TOK, EMB)


# speedup vs baseline: 1.1101x; 1.1101x over previous
import functools

import jax
import jax.numpy as jnp
from jax import lax
from jax.experimental import pallas as pl
from jax.experimental.pallas import tpu as pltpu
from jax.experimental.pallas import tpu_sc as plsc

B_TOK = 4096
T_TOK = 200
EMB = 64
NW = 32
B = B_TOK * T_TOK
B_PER_W = B // NW
CH = 128
NCH = B_PER_W // CH

_NC = 2
NBUF = 4
_MESH = plsc.VectorSubcoreMesh(core_axis_name="c", subcore_axis_name="s")


@functools.partial(
    pl.kernel,
    mesh=_MESH,
    out_type=jax.ShapeDtypeStruct((B, EMB), jnp.float32),
    scratch_types=[
        pltpu.VMEM((NCH, CH), jnp.int32),
        pltpu.VMEM((NBUF, CH, EMB), jnp.float32),
        [pltpu.SemaphoreType.DMA] * NBUF,
        [pltpu.SemaphoreType.DMA] * NBUF,
    ],
    compiler_params=pltpu.CompilerParams(use_tc_tiling_on_sc=False),
)
def _gather_kernel(tok_hbm, table_hbm, out_hbm, idx_v, rows_v, gsems, wsems):
    wid = lax.axis_index("s") * _NC + lax.axis_index("c")
    base = wid * B_PER_W
    pltpu.sync_copy(tok_hbm.at[wid], idx_v)

    def gather(j, b):
        pltpu.async_copy(table_hbm.at[idx_v.at[j]], rows_v.at[b], gsems[b])

    def write(j, b):
        pltpu.async_copy(rows_v.at[b], out_hbm.at[pl.ds(base + j * CH, CH)],
                         wsems[b])

    def wait_gather(j, b):
        pltpu.make_async_copy(table_hbm.at[idx_v.at[b]], rows_v.at[b],
                              gsems[b]).wait()

    def wait_write(j, b):
        pltpu.make_async_copy(rows_v.at[b],
                              out_hbm.at[pl.ds(base + j * CH, CH)],
                              wsems[b]).wait()

    for b in range(NBUF):
        gather(b, b)

    def group(g, carry):
        for b in range(NBUF):
            j = g * NBUF + b
            wait_gather(j, b)
            write(j, b)
        for b in range(NBUF):
            j = g * NBUF + b
            wait_write(j, b)
            gather(j + NBUF, b)
        return carry

    lax.fori_loop(0, NCH // NBUF - 1, group, 0)

    for b in range(NBUF):
        j = NCH - NBUF + b
        wait_gather(j, b)
        write(j, b)
    for b in range(NBUF):
        j = NCH - NBUF + b
        wait_write(j, b)


def kernel(tokens, table):
    tok = tokens.reshape(NW, NCH, CH)
    out = _gather_kernel(tok, table)
    return out.reshape(B_TOK, T_TOK, EMB)


# speedup vs baseline: 1.3575x; 1.2228x over previous
import functools

import jax
import jax.numpy as jnp
from jax import lax
from jax.experimental import pallas as pl
from jax.experimental.pallas import tpu as pltpu
from jax.experimental.pallas import tpu_sc as plsc

B_TOK = 4096
T_TOK = 200
EMB = 64
PEMB = 128
NW = 32
B = B_TOK * T_TOK
B_PER_W = B // NW
CH = 128
NCH = B_PER_W // CH

_NC = 2
NBUF = 4
_MESH = plsc.VectorSubcoreMesh(core_axis_name="c", subcore_axis_name="s")


@functools.partial(
    pl.kernel,
    mesh=_MESH,
    out_type=jax.ShapeDtypeStruct((B, PEMB), jnp.float32),
    scratch_types=[
        pltpu.VMEM((NCH, CH), jnp.int32),
        pltpu.VMEM((NBUF, CH, PEMB), jnp.float32),
        [pltpu.SemaphoreType.DMA] * NBUF,
        [pltpu.SemaphoreType.DMA] * NBUF,
    ],
    compiler_params=pltpu.CompilerParams(use_tc_tiling_on_sc=True),
)
def _gather_kernel(tok_hbm, table_hbm, out_hbm, idx_v, rows_v, gsems, wsems):
    wid = lax.axis_index("s") * _NC + lax.axis_index("c")
    base = wid * B_PER_W
    pltpu.sync_copy(tok_hbm.at[wid], idx_v)

    def gather(j, b):
        pltpu.async_copy(table_hbm.at[idx_v.at[j]], rows_v.at[b], gsems[b])

    def write(j, b):
        pltpu.async_copy(rows_v.at[b], out_hbm.at[pl.ds(base + j * CH, CH)],
                         wsems[b])

    def wait_gather(j, b):
        pltpu.make_async_copy(table_hbm.at[idx_v.at[b]], rows_v.at[b],
                              gsems[b]).wait()

    def wait_write(j, b):
        pltpu.make_async_copy(rows_v.at[b],
                              out_hbm.at[pl.ds(base + j * CH, CH)],
                              wsems[b]).wait()

    for b in range(NBUF):
        gather(b, b)

    def group(g, carry):
        for b in range(NBUF):
            j = g * NBUF + b
            wait_gather(j, b)
            write(j, b)
        for b in range(NBUF):
            j = g * NBUF + b
            wait_write(j, b)
            gather(j + NBUF, b)
        return carry

    lax.fori_loop(0, NCH // NBUF - 1, group, 0)

    for b in range(NBUF):
        j = NCH - NBUF + b
        wait_gather(j, b)
        write(j, b)
    for b in range(NBUF):
        j = NCH - NBUF + b
        wait_write(j, b)


def kernel(tokens, table):
    tok = tokens.reshape(NW, NCH, CH)
    tab = jnp.pad(table, ((0, 0), (0, PEMB - EMB)))
    out = _gather_kernel(tok, tab)
    return out[:, :EMB].reshape(B_TOK, T_TOK, EMB)
